# baseline (speedup 1.0000x reference)
import functools

import jax
import jax.numpy as jnp
from jax import lax
from jax.experimental import pallas as pl
from jax.experimental.pallas import tpu as pltpu
from jax.experimental.pallas import tpu_sc as plsc

N_NODES = 10000
N_PAD = 10240
HDIM = 64
NC, NS = 2, 16
NW = NC * NS
CHUNK = 512
KROWS = CHUNK // 128
ROWS_PER_SUB = N_PAD // NS
NPARTS = 5
QUARTER = ROWS_PER_SUB // NPARTS
LANES = 16


def _bcast_lane(v, k):
    idx = jnp.full((LANES, 1), k, jnp.int32)
    return lax.gather(
        v, idx,
        lax.GatherDimensionNumbers(offset_dims=(), collapsed_slice_dims=(0,),
                                   start_index_map=(0,)),
        (1,), mode=lax.GatherScatterMode.PROMISE_IN_BOUNDS)


def _make_spmm(e_total, fuse_relu):
    epw = -(-e_total // NW)
    epw = -(-epw // CHUNK) * CHUNK
    n_chunks = epw // CHUNK
    rem = e_total - epw * (NW - 1)
    assert rem > 0 and rem % CHUNK == 0 and epw % 128 == 0
    nc_last = rem // CHUNK
    erows_pw = epw // 128
    mesh = plsc.VectorSubcoreMesh(core_axis_name="c", subcore_axis_name="s")

    if fuse_relu:
        out_type = [jax.ShapeDtypeStruct((NC, N_PAD, HDIM), jnp.float32),
                    jax.ShapeDtypeStruct((NC, N_PAD, HDIM), jnp.float32)]
    else:
        out_type = jax.ShapeDtypeStruct((NC, N_PAD, HDIM), jnp.float32)

    @functools.partial(
        pl.kernel,
        out_type=out_type,
        mesh=mesh,
        compiler_params=pltpu.CompilerParams(use_tc_tiling_on_sc=False),
        scratch_types=[
            pltpu.VMEM((3, KROWS, 128), jnp.int32),
            pltpu.VMEM((3, KROWS, 128), jnp.int32),
            pltpu.VMEM((3, CHUNK), jnp.float32),
            pltpu.VMEM((2, CHUNK, HDIM), jnp.float32),
            pltpu.VMEM((2, QUARTER, HDIM), jnp.float32),
            pltpu.VMEM_SHARED((N_PAD, HDIM), jnp.float32),
            pltpu.SemaphoreType.DMA,
            pltpu.SemaphoreType.DMA,
            pltpu.SemaphoreType.DMA,
        ],
    )
    def spmm(h_hbm, ei_hbm, w_hbm, *outs_and_scratch):
        if fuse_relu:
            (out_hbm, hid_hbm, idx_s, idx_d, wbuf, rows, pbuf, acc,
             sem_g, sem_i, sem_s) = outs_and_scratch
        else:
            (out_hbm, idx_s, idx_d, wbuf, rows, pbuf, acc,
             sem_g, sem_i, sem_s) = outs_and_scratch
        cid = lax.axis_index("c")
        sid = lax.axis_index("s")
        wid = cid * NS + sid
        my_chunks = jnp.where(wid == NW - 1, nc_last, n_chunks)

        zeros16 = jnp.zeros((LANES,), jnp.float32)

        @plsc.parallel_loop(0, CHUNK * (HDIM // LANES))
        def _zero(t):
            rows[0, t // (HDIM // LANES),
                 pl.ds((t % (HDIM // LANES)) * LANES, LANES)] = zeros16

        base_row = sid * ROWS_PER_SUB
        for part in range(ROWS_PER_SUB // CHUNK):
            pltpu.sync_copy(rows.at[0],
                            acc.at[pl.ds(base_row + part * CHUNK, CHUNK)])
        tail = ROWS_PER_SUB % CHUNK
        if tail:
            pltpu.sync_copy(rows.at[0, pl.ds(0, tail)],
                            acc.at[pl.ds(base_row + ROWS_PER_SUB - tail, tail)])

        if fuse_relu:
            def fire_part(part):
                o = base_row + part * QUARTER
                b = part % 2
                pltpu.async_copy(h_hbm.at[0, pl.ds(o, QUARTER)],
                                 rows.at[b, pl.ds(0, QUARTER)], sem_g)
                pltpu.async_copy(h_hbm.at[1, pl.ds(o, QUARTER)],
                                 pbuf.at[b], sem_g)

            def wait_store():
                pltpu.make_async_copy(rows.at[0, pl.ds(0, QUARTER)],
                                      hid_hbm.at[cid, pl.ds(base_row, QUARTER)],
                                      sem_s).wait()

            fire_part(0)
            for part in range(NPARTS):
                o = base_row + part * QUARTER
                b = part % 2
                pltpu.make_async_copy(h_hbm.at[0, pl.ds(0, QUARTER)],
                                      rows.at[b, pl.ds(0, QUARTER)],
                                      sem_g).wait()
                pltpu.make_async_copy(h_hbm.at[1, pl.ds(0, QUARTER)],
                                      pbuf.at[b], sem_g).wait()
                if part + 1 < NPARTS:
                    if part >= 1:
                        wait_store()
                    fire_part(part + 1)

                @plsc.parallel_loop(0, QUARTER * (HDIM // LANES))
                def _rc(t):
                    r = t // (HDIM // LANES)
                    c = pl.ds((t % (HDIM // LANES)) * LANES, LANES)
                    rows[b, r, c] = jnp.maximum(rows[b, r, c] + pbuf[b, r, c],
                                                0.0)

                pltpu.async_copy(rows.at[b, pl.ds(0, QUARTER)],
                                 hid_hbm.at[cid, pl.ds(o, QUARTER)], sem_s)
            wait_store()
            wait_store()
            gsrc = hid_hbm.at[cid]
        else:
            gsrc = h_hbm
        plsc.subcore_barrier()

        def fire_idx(ci, buf):
            r0 = wid * erows_pw + ci * KROWS
            pltpu.async_copy(ei_hbm.at[1, pl.ds(r0, KROWS)], idx_s.at[buf], sem_i)
            pltpu.async_copy(ei_hbm.at[0, pl.ds(r0, KROWS)], idx_d.at[buf], sem_i)
            pltpu.async_copy(w_hbm.at[pl.ds(wid * epw + ci * CHUNK, CHUNK)],
                             wbuf.at[buf], sem_i)

        def wait_idx(buf):
            pltpu.make_async_copy(ei_hbm.at[1, pl.ds(0, KROWS)],
                                  idx_s.at[buf], sem_i).wait()
            pltpu.make_async_copy(ei_hbm.at[0, pl.ds(0, KROWS)],
                                  idx_d.at[buf], sem_i).wait()
            pltpu.make_async_copy(w_hbm.at[pl.ds(0, CHUNK)],
                                  wbuf.at[buf], sem_i).wait()

        def fire_gathers(rbuf, ibuf):
            for j in range(KROWS):
                pltpu.async_copy(gsrc.at[idx_s.at[ibuf, j]],
                                 rows.at[rbuf, pl.ds(j * 128, 128)], sem_g)

        def wait_gathers(rbuf, ibuf):
            for j in range(KROWS):
                pltpu.make_async_copy(gsrc.at[idx_s.at[ibuf, j]],
                                      rows.at[rbuf, pl.ds(j * 128, 128)],
                                      sem_g).wait()

        fire_idx(0, 0)
        wait_idx(0)
        fire_gathers(0, 0)
        fire_idx(1, 1)

        def wait_scatters(rbuf):
            for j in range(KROWS):
                pltpu.make_async_copy(rows.at[rbuf, pl.ds(j * 128, 128)],
                                      acc.at[idx_d.at[0, j]], sem_s).wait()

        def chunk_body(ci, carry):
            rcur = lax.rem(ci, 2)
            rnxt = 1 - rcur
            bcur = lax.rem(ci, 3)
            bn1 = lax.rem(ci + 1, 3)
            bn2 = lax.rem(ci + 2, 3)

            @pl.when(ci > 0)
            def _():
                wait_scatters(rnxt)

            @pl.when(ci + 1 < my_chunks)
            def _():
                wait_idx(bn1)
                fire_gathers(rnxt, bn1)

            @pl.when(ci + 2 < my_chunks)
            def _():
                fire_idx(ci + 2, bn2)

            wait_gathers(rcur, bcur)

            @plsc.parallel_loop(0, CHUNK // LANES, unroll=2)
            def _scale(g):
                wv16 = wbuf[bcur, pl.ds(g * LANES, LANES)]
                for k in range(LANES):
                    wv = _bcast_lane(wv16, k)
                    e = g * LANES + k
                    for j in range(HDIM // LANES):
                        sl = pl.ds(j * LANES, LANES)
                        rows[rcur, e, sl] = rows[rcur, e, sl] * wv

            for j in range(KROWS):
                pltpu.async_copy(rows.at[rcur, pl.ds(j * 128, 128)],
                                 acc.at[idx_d.at[bcur, j]], sem_s, add=True)
            return carry

        lax.fori_loop(0, my_chunks, chunk_body, 0)
        wait_scatters(lax.rem(my_chunks - 1, 2))
        plsc.subcore_barrier()
        pltpu.sync_copy(acc.at[pl.ds(sid * ROWS_PER_SUB, ROWS_PER_SUB)],
                        out_hbm.at[cid, pl.ds(sid * ROWS_PER_SUB, ROWS_PER_SUB)])

    return spmm


def _mm_body(x_ref, w_ref, o_ref):
    o_ref[...] = jnp.dot(x_ref[...], w_ref[...],
                         preferred_element_type=jnp.float32)


def _tc_matmul(x, w):
    n, f = x.shape
    h = w.shape[1]
    blk = 2000
    return pl.pallas_call(
        _mm_body,
        grid=(n // blk,),
        in_specs=[pl.BlockSpec((blk, f), lambda i: (i, 0)),
                  pl.BlockSpec((f, h), lambda i: (0, 0))],
        out_specs=pl.BlockSpec((blk, h), lambda i: (i, 0)),
        out_shape=jax.ShapeDtypeStruct((n, h), jnp.float32),
    )(x, w)


def _final_body(q_ref, wm_ref, ws_ref, noise_ref, o_ref):
    q = q_ref[...]
    s = q[0] + q[1]
    mean = jnp.dot(s, wm_ref[...], preferred_element_type=jnp.float32)
    log_std = jnp.dot(s, ws_ref[...], preferred_element_type=jnp.float32)
    o_ref[...] = mean + log_std * noise_ref[...]


def _final(q, wm, ws, noise):
    _, _, h = q.shape
    n = N_NODES
    z = wm.shape[1]
    blk = 2000
    return pl.pallas_call(
        _final_body,
        grid=(n // blk,),
        in_specs=[pl.BlockSpec((2, blk, h), lambda i: (0, i, 0)),
                  pl.BlockSpec((h, z), lambda i: (0, 0)),
                  pl.BlockSpec((h, z), lambda i: (0, 0)),
                  pl.BlockSpec((blk, z), lambda i: (i, 0))],
        out_specs=pl.BlockSpec((blk, z), lambda i: (i, 0)),
        out_shape=jax.ShapeDtypeStruct((n, z), jnp.float32),
    )(q, wm, ws, noise)


def kernel(x, edge_index, edge_weight, W0, Wm, Ws, noise):
    e = edge_index.shape[1]
    ei = edge_index.reshape(2, e // 128, 128)
    spmm1 = _make_spmm(e, fuse_relu=False)
    spmm2 = _make_spmm(e, fuse_relu=True)

    h0 = _tc_matmul(x, W0)
    p = spmm1(h0, ei, edge_weight)
    q, _ = spmm2(p, ei, edge_weight)
    return _final(q, Wm, Ws, noise)


# speedup vs baseline: 1.0664x; 1.0664x over previous
import functools

import jax
import jax.numpy as jnp
from jax import lax
from jax.experimental import pallas as pl
from jax.experimental.pallas import tpu as pltpu
from jax.experimental.pallas import tpu_sc as plsc

N_NODES = 10000
N_PAD = 10240
HDIM = 64
NC, NS = 2, 16
NW = NC * NS
CHUNK = 512
KROWS = CHUNK // 128
ROWS_PER_SUB = N_PAD // NS
NPARTS = 5
QUARTER = ROWS_PER_SUB // NPARTS
LANES = 16


def _bcast_lane(v, k):
    idx = jnp.full((LANES, 1), k, jnp.int32)
    return lax.gather(
        v, idx,
        lax.GatherDimensionNumbers(offset_dims=(), collapsed_slice_dims=(0,),
                                   start_index_map=(0,)),
        (1,), mode=lax.GatherScatterMode.PROMISE_IN_BOUNDS)


def _make_spmm(e_total, fuse_relu):
    epw = -(-e_total // NW)
    epw = -(-epw // CHUNK) * CHUNK
    n_chunks = epw // CHUNK
    rem = e_total - epw * (NW - 1)
    assert rem > 0 and rem % CHUNK == 0 and epw % 128 == 0
    nc_last = rem // CHUNK
    erows_pw = epw // 128
    mesh = plsc.VectorSubcoreMesh(core_axis_name="c", subcore_axis_name="s")

    if fuse_relu:
        out_type = [jax.ShapeDtypeStruct((NC, N_PAD, HDIM), jnp.float32),
                    jax.ShapeDtypeStruct((NC, N_PAD, HDIM), jnp.float32)]
    else:
        out_type = jax.ShapeDtypeStruct((NC, N_PAD, HDIM), jnp.float32)

    @functools.partial(
        pl.kernel,
        out_type=out_type,
        mesh=mesh,
        compiler_params=pltpu.CompilerParams(use_tc_tiling_on_sc=False),
        scratch_types=[
            pltpu.VMEM((3, KROWS, 128), jnp.int32),
            pltpu.VMEM((3, KROWS, 128), jnp.int32),
            pltpu.VMEM((3, CHUNK), jnp.float32),
            pltpu.VMEM((2, CHUNK, HDIM), jnp.float32),
            pltpu.VMEM((2, QUARTER, HDIM), jnp.float32),
            pltpu.VMEM_SHARED((N_PAD, HDIM), jnp.float32),
            pltpu.SemaphoreType.DMA,
            pltpu.SemaphoreType.DMA,
            pltpu.SemaphoreType.DMA,
        ],
    )
    def spmm(h_hbm, ei_hbm, w_hbm, *outs_and_scratch):
        if fuse_relu:
            (out_hbm, hid_hbm, idx_s, idx_d, wbuf, rows, pbuf, acc,
             sem_g, sem_i, sem_s) = outs_and_scratch
        else:
            (out_hbm, idx_s, idx_d, wbuf, rows, pbuf, acc,
             sem_g, sem_i, sem_s) = outs_and_scratch
        cid = lax.axis_index("c")
        sid = lax.axis_index("s")
        wid = cid * NS + sid
        my_chunks = jnp.where(wid == NW - 1, nc_last, n_chunks)

        zeros16 = jnp.zeros((LANES,), jnp.float32)

        @plsc.parallel_loop(0, CHUNK * (HDIM // LANES))
        def _zero(t):
            rows[0, t // (HDIM // LANES),
                 pl.ds((t % (HDIM // LANES)) * LANES, LANES)] = zeros16

        base_row = sid * ROWS_PER_SUB
        for part in range(ROWS_PER_SUB // CHUNK):
            pltpu.sync_copy(rows.at[0],
                            acc.at[pl.ds(base_row + part * CHUNK, CHUNK)])
        tail = ROWS_PER_SUB % CHUNK
        if tail:
            pltpu.sync_copy(rows.at[0, pl.ds(0, tail)],
                            acc.at[pl.ds(base_row + ROWS_PER_SUB - tail, tail)])

        if fuse_relu:
            def fire_part(part):
                o = base_row + part * QUARTER
                b = part % 2
                pltpu.async_copy(h_hbm.at[0, pl.ds(o, QUARTER)],
                                 rows.at[b, pl.ds(0, QUARTER)], sem_g)
                pltpu.async_copy(h_hbm.at[1, pl.ds(o, QUARTER)],
                                 pbuf.at[b], sem_g)

            def wait_store():
                pltpu.make_async_copy(rows.at[0, pl.ds(0, QUARTER)],
                                      hid_hbm.at[cid, pl.ds(base_row, QUARTER)],
                                      sem_s).wait()

            fire_part(0)
            for part in range(NPARTS):
                o = base_row + part * QUARTER
                b = part % 2
                pltpu.make_async_copy(h_hbm.at[0, pl.ds(0, QUARTER)],
                                      rows.at[b, pl.ds(0, QUARTER)],
                                      sem_g).wait()
                pltpu.make_async_copy(h_hbm.at[1, pl.ds(0, QUARTER)],
                                      pbuf.at[b], sem_g).wait()
                if part + 1 < NPARTS:
                    if part >= 1:
                        wait_store()
                    fire_part(part + 1)

                @plsc.parallel_loop(0, QUARTER * (HDIM // LANES))
                def _rc(t):
                    r = t // (HDIM // LANES)
                    c = pl.ds((t % (HDIM // LANES)) * LANES, LANES)
                    rows[b, r, c] = jnp.maximum(rows[b, r, c] + pbuf[b, r, c],
                                                0.0)

                pltpu.async_copy(rows.at[b, pl.ds(0, QUARTER)],
                                 hid_hbm.at[cid, pl.ds(o, QUARTER)], sem_s)
            wait_store()
            wait_store()
            gsrc = hid_hbm.at[cid]
        else:
            gsrc = h_hbm
        plsc.subcore_barrier()

        def fire_idx(ci, buf):
            r0 = wid * erows_pw + ci * KROWS
            pltpu.async_copy(ei_hbm.at[1, pl.ds(r0, KROWS)], idx_s.at[buf], sem_i)
            pltpu.async_copy(ei_hbm.at[0, pl.ds(r0, KROWS)], idx_d.at[buf], sem_i)
            pltpu.async_copy(w_hbm.at[pl.ds(wid * epw + ci * CHUNK, CHUNK)],
                             wbuf.at[buf], sem_i)

        def wait_idx(buf):
            pltpu.make_async_copy(ei_hbm.at[1, pl.ds(0, KROWS)],
                                  idx_s.at[buf], sem_i).wait()
            pltpu.make_async_copy(ei_hbm.at[0, pl.ds(0, KROWS)],
                                  idx_d.at[buf], sem_i).wait()
            pltpu.make_async_copy(w_hbm.at[pl.ds(0, CHUNK)],
                                  wbuf.at[buf], sem_i).wait()

        def fire_gathers(rbuf, ibuf):
            for j in range(KROWS):
                pltpu.async_copy(gsrc.at[idx_s.at[ibuf, j]],
                                 rows.at[rbuf, pl.ds(j * 128, 128)], sem_g)

        def wait_gathers(rbuf, ibuf):
            for j in range(KROWS):
                pltpu.make_async_copy(gsrc.at[idx_s.at[ibuf, j]],
                                      rows.at[rbuf, pl.ds(j * 128, 128)],
                                      sem_g).wait()

        fire_idx(0, 0)
        wait_idx(0)
        fire_gathers(0, 0)
        fire_idx(1, 1)

        def wait_scatters(rbuf):
            for j in range(KROWS):
                pltpu.make_async_copy(rows.at[rbuf, pl.ds(j * 128, 128)],
                                      acc.at[idx_d.at[0, j]], sem_s).wait()

        def chunk_body(ci, carry):
            rcur = lax.rem(ci, 2)
            rnxt = 1 - rcur
            bcur = lax.rem(ci, 3)
            bn1 = lax.rem(ci + 1, 3)
            bn2 = lax.rem(ci + 2, 3)

            @pl.when(ci > 0)
            def _():
                wait_scatters(rnxt)

            @pl.when(ci + 1 < my_chunks)
            def _():
                wait_idx(bn1)
                fire_gathers(rnxt, bn1)

            @pl.when(ci + 2 < my_chunks)
            def _():
                fire_idx(ci + 2, bn2)

            wait_gathers(rcur, bcur)

            @plsc.parallel_loop(0, CHUNK // LANES)
            def _scale(g):
                wv16 = wbuf[bcur, pl.ds(g * LANES, LANES)]
                for k in range(LANES):
                    wv = _bcast_lane(wv16, k)
                    e = g * LANES + k
                    for j in range(HDIM // LANES):
                        sl = pl.ds(j * LANES, LANES)
                        rows[rcur, e, sl] = rows[rcur, e, sl] * wv

            for j in range(KROWS):
                pltpu.async_copy(rows.at[rcur, pl.ds(j * 128, 128)],
                                 acc.at[idx_d.at[bcur, j]], sem_s, add=True)
            return carry

        lax.fori_loop(0, my_chunks, chunk_body, 0)
        wait_scatters(lax.rem(my_chunks - 1, 2))
        plsc.subcore_barrier()
        pltpu.sync_copy(acc.at[pl.ds(sid * ROWS_PER_SUB, ROWS_PER_SUB)],
                        out_hbm.at[cid, pl.ds(sid * ROWS_PER_SUB, ROWS_PER_SUB)])

    return spmm


def _mm_body(x_ref, w_ref, o_ref):
    o_ref[...] = jnp.dot(x_ref[...], w_ref[...],
                         preferred_element_type=jnp.float32)


def _tc_matmul(x, w):
    n, f = x.shape
    h = w.shape[1]
    blk = 2000
    return pl.pallas_call(
        _mm_body,
        grid=(n // blk,),
        in_specs=[pl.BlockSpec((blk, f), lambda i: (i, 0)),
                  pl.BlockSpec((f, h), lambda i: (0, 0))],
        out_specs=pl.BlockSpec((blk, h), lambda i: (i, 0)),
        out_shape=jax.ShapeDtypeStruct((n, h), jnp.float32),
    )(x, w)


def _final_body(q_ref, wm_ref, ws_ref, noise_ref, o_ref):
    q = q_ref[...]
    s = q[0] + q[1]
    mean = jnp.dot(s, wm_ref[...], preferred_element_type=jnp.float32)
    log_std = jnp.dot(s, ws_ref[...], preferred_element_type=jnp.float32)
    o_ref[...] = mean + log_std * noise_ref[...]


def _final(q, wm, ws, noise):
    _, _, h = q.shape
    n = N_NODES
    z = wm.shape[1]
    blk = 2000
    return pl.pallas_call(
        _final_body,
        grid=(n // blk,),
        in_specs=[pl.BlockSpec((2, blk, h), lambda i: (0, i, 0)),
                  pl.BlockSpec((h, z), lambda i: (0, 0)),
                  pl.BlockSpec((h, z), lambda i: (0, 0)),
                  pl.BlockSpec((blk, z), lambda i: (i, 0))],
        out_specs=pl.BlockSpec((blk, z), lambda i: (i, 0)),
        out_shape=jax.ShapeDtypeStruct((n, z), jnp.float32),
    )(q, wm, ws, noise)


def kernel(x, edge_index, edge_weight, W0, Wm, Ws, noise):
    e = edge_index.shape[1]
    ei = edge_index.reshape(2, e // 128, 128)
    spmm1 = _make_spmm(e, fuse_relu=False)
    spmm2 = _make_spmm(e, fuse_relu=True)

    h0 = _tc_matmul(x, W0)
    p = spmm1(h0, ei, edge_weight)
    q, _ = spmm2(p, ei, edge_weight)
    return _final(q, Wm, Ws, noise)
